# baseline (speedup 1.0000x reference)
import jax
import jax.numpy as jnp
from jax.experimental import pallas as pl
from jax.experimental.pallas import tpu as pltpu

_B = 8
_C_IN = 3
_H = 384
_W = 384
_P = 16
_N_TOK = (_H // _P) * (_W // _P)
_CP = _C_IN * _P * _P
_C = 384
_E = 4
_THR = 0.1


def _moe_step(x_ref, Wps_ref, W_desc_ref, W_router_ref, W_all_ref,
              out_ref, pbuf_ref):
    i = pl.program_id(0)

    @pl.when(i > 0)
    def _compute():
        patches = pbuf_ref[(i + 1) % 2]
        both = jnp.dot(patches, Wps_ref[...],
                       preferred_element_type=jnp.float32)
        feat = jnp.tanh(both[:, :_C])
        skip = both[:, _C:]

        mean = jnp.mean(feat, axis=0, keepdims=True)
        desc = jnp.dot(mean, W_desc_ref[...],
                       preferred_element_type=jnp.float32)
        logits = jnp.dot(desc, W_router_ref[...],
                         preferred_element_type=jnp.float32)
        lmax = jnp.max(logits)
        conf = 1.0 / jnp.sum(jnp.exp(logits - lmax))
        eid = jnp.argmax(logits[0]).astype(jnp.int32)
        idx = jnp.where(conf < _THR, jnp.int32(_E), eid)

        acc = jnp.dot(feat, W_all_ref[idx],
                      preferred_element_type=jnp.float32)
        out_ref[0] = jnp.maximum(acc, 0.0) + skip

    @pl.when(i < _B)
    def _relayout():
        v = x_ref[0].reshape(_C_IN, _H // _P, _P, _W // _P, _P)
        pbuf_ref[i % 2] = v.transpose(1, 3, 0, 2, 4).reshape(_N_TOK, _CP)


@jax.jit
def kernel(x, W_patch, W_skip, W_desc, W_router, W_experts, W_fallback):
    Wps = jnp.concatenate([W_patch, W_skip], axis=1)
    W_all = jnp.concatenate([W_experts, W_fallback[None]], 0)

    grid_spec = dict(
        grid=(_B + 1,),
        in_specs=[
            pl.BlockSpec((1, _C_IN, _H, _W),
                         lambda i: (jnp.minimum(i, _B - 1), 0, 0, 0)),
            pl.BlockSpec((_CP, 2 * _C), lambda i: (0, 0)),
            pl.BlockSpec((_C, _C), lambda i: (0, 0)),
            pl.BlockSpec((_C, _E), lambda i: (0, 0)),
            pl.BlockSpec((_E + 1, _C, _C), lambda i: (0, 0, 0)),
        ],
        out_specs=pl.BlockSpec((1, _N_TOK, _C),
                               lambda i: (jnp.maximum(i - 1, 0), 0, 0)),
    )
    return pl.pallas_call(
        _moe_step,
        out_shape=jax.ShapeDtypeStruct((_B, _N_TOK, _C), jnp.float32),
        scratch_shapes=[pltpu.VMEM((2, _N_TOK, _CP), jnp.float32)],
        **grid_spec,
    )(x, Wps, W_desc, W_router, W_all)


# speedup vs baseline: 1.0888x; 1.0888x over previous
import jax
import jax.numpy as jnp
from jax.experimental import pallas as pl
from jax.experimental.pallas import tpu as pltpu

_B = 8
_C_IN = 3
_H = 384
_W = 384
_P = 16
_N_TOK = (_H // _P) * (_W // _P)
_CP = _C_IN * _P * _P
_C = 384
_E = 4
_THR = 0.1


def _moe_step(x_ref, W_patch_ref, W_skip_ref, W_desc_ref, W_router_ref,
              W_all_ref, out_ref):
    v = x_ref[0].reshape(_C_IN, _H // _P, _P, _W // _P, _P)
    patches = v.transpose(1, 3, 0, 2, 4).reshape(_N_TOK, _CP)

    feat = jnp.tanh(jnp.dot(patches, W_patch_ref[...],
                            preferred_element_type=jnp.float32))
    skip = jnp.dot(patches.astype(jnp.bfloat16), W_skip_ref[...],
                   preferred_element_type=jnp.float32)

    mean = jnp.mean(feat, axis=0, keepdims=True)
    desc = jnp.dot(mean, W_desc_ref[...],
                   preferred_element_type=jnp.float32)
    logits = jnp.dot(desc, W_router_ref[...],
                     preferred_element_type=jnp.float32)
    lmax = jnp.max(logits)
    conf = 1.0 / jnp.sum(jnp.exp(logits - lmax))
    eid = jnp.argmax(logits[0]).astype(jnp.int32)
    idx = jnp.where(conf < _THR, jnp.int32(_E), eid)

    acc = jnp.dot(feat.astype(jnp.bfloat16), W_all_ref[idx],
                  preferred_element_type=jnp.float32)
    out_ref[0] = jnp.maximum(acc, 0.0) + skip


@jax.jit
def kernel(x, W_patch, W_skip, W_desc, W_router, W_experts, W_fallback):
    W_skip_bf = W_skip.astype(jnp.bfloat16)
    W_all = jnp.concatenate(
        [W_experts, W_fallback[None]], 0).astype(jnp.bfloat16)

    grid_spec = dict(
        grid=(_B,),
        in_specs=[
            pl.BlockSpec((1, _C_IN, _H, _W), lambda b: (b, 0, 0, 0)),
            pl.BlockSpec((_CP, _C), lambda b: (0, 0)),
            pl.BlockSpec((_CP, _C), lambda b: (0, 0)),
            pl.BlockSpec((_C, _C), lambda b: (0, 0)),
            pl.BlockSpec((_C, _E), lambda b: (0, 0)),
            pl.BlockSpec((_E + 1, _C, _C), lambda b: (0, 0, 0)),
        ],
        out_specs=pl.BlockSpec((1, _N_TOK, _C), lambda b: (b, 0, 0)),
    )
    return pl.pallas_call(
        _moe_step,
        out_shape=jax.ShapeDtypeStruct((_B, _N_TOK, _C), jnp.float32),
        **grid_spec,
    )(x, W_patch, W_skip_bf, W_desc, W_router, W_all)


# speedup vs baseline: 1.4870x; 1.3657x over previous
import jax
import jax.numpy as jnp
from jax.experimental import pallas as pl
from jax.experimental.pallas import tpu as pltpu

_B = 8
_C_IN = 3
_H = 384
_W = 384
_P = 16
_N_TOK = (_H // _P) * (_W // _P)
_CP = _C_IN * _P * _P
_C = 384
_E = 4
_THR = 0.1


def _moe_step(x_ref, W_patch_ref, W_skip_ref, W_desc_ref, W_router_ref,
              W_all_ref, out_ref):
    v = x_ref[0].astype(jnp.bfloat16).reshape(_C_IN, _H // _P, _P,
                                              _W // _P, _P)
    patches = v.transpose(1, 3, 0, 2, 4).reshape(_N_TOK, _CP)

    feat = jnp.tanh(jnp.dot(patches, W_patch_ref[...],
                            preferred_element_type=jnp.float32))
    skip = jnp.dot(patches, W_skip_ref[...],
                   preferred_element_type=jnp.float32)

    mean = jnp.mean(feat, axis=0, keepdims=True)
    desc = jnp.dot(mean, W_desc_ref[...],
                   preferred_element_type=jnp.float32)
    logits = jnp.dot(desc, W_router_ref[...],
                     preferred_element_type=jnp.float32)
    lmax = jnp.max(logits)
    conf = 1.0 / jnp.sum(jnp.exp(logits - lmax))
    eid = jnp.argmax(logits[0]).astype(jnp.int32)
    idx = jnp.where(conf < _THR, jnp.int32(_E), eid)

    acc = jnp.dot(feat.astype(jnp.bfloat16), W_all_ref[idx],
                  preferred_element_type=jnp.float32)
    out_ref[0] = jnp.maximum(acc, 0.0) + skip


@jax.jit
def kernel(x, W_patch, W_skip, W_desc, W_router, W_experts, W_fallback):
    W_patch_bf = W_patch.astype(jnp.bfloat16)
    W_skip_bf = W_skip.astype(jnp.bfloat16)
    W_all = jnp.concatenate(
        [W_experts, W_fallback[None]], 0).astype(jnp.bfloat16)

    grid_spec = dict(
        grid=(_B,),
        in_specs=[
            pl.BlockSpec((1, _C_IN, _H, _W), lambda b: (b, 0, 0, 0)),
            pl.BlockSpec((_CP, _C), lambda b: (0, 0)),
            pl.BlockSpec((_CP, _C), lambda b: (0, 0)),
            pl.BlockSpec((_C, _C), lambda b: (0, 0)),
            pl.BlockSpec((_C, _E), lambda b: (0, 0)),
            pl.BlockSpec((_E + 1, _C, _C), lambda b: (0, 0, 0)),
        ],
        out_specs=pl.BlockSpec((1, _N_TOK, _C), lambda b: (b, 0, 0)),
    )
    return pl.pallas_call(
        _moe_step,
        out_shape=jax.ShapeDtypeStruct((_B, _N_TOK, _C), jnp.float32),
        **grid_spec,
    )(x, W_patch_bf, W_skip_bf, W_desc, W_router, W_all)


# speedup vs baseline: 1.4927x; 1.0039x over previous
import jax
import jax.numpy as jnp
from jax.experimental import pallas as pl
from jax.experimental.pallas import tpu as pltpu

_B = 8
_C_IN = 3
_H = 384
_W = 384
_P = 16
_N_TOK = (_H // _P) * (_W // _P)
_CP = _C_IN * _P * _P
_C = 384
_E = 4
_THR = 0.1


def _moe_step(x_ref, W_ps_ref, W_desc_ref, W_router_ref,
              W_all_ref, out_ref):
    v = x_ref[0].astype(jnp.bfloat16).reshape(_C_IN, _H // _P, _P,
                                              _W // _P, _P)
    patches = v.transpose(1, 3, 0, 2, 4).reshape(_N_TOK, _CP)

    both = jnp.dot(patches, W_ps_ref[...],
                   preferred_element_type=jnp.float32)
    feat = jnp.tanh(both[:, :_C])
    skip = both[:, _C:]

    mean = jnp.mean(feat, axis=0, keepdims=True)
    desc = jnp.dot(mean, W_desc_ref[...],
                   preferred_element_type=jnp.float32)
    logits = jnp.dot(desc, W_router_ref[...],
                     preferred_element_type=jnp.float32)
    lmax = jnp.max(logits)
    conf = 1.0 / jnp.sum(jnp.exp(logits - lmax))
    eid = jnp.argmax(logits[0]).astype(jnp.int32)
    idx = jnp.where(conf < _THR, jnp.int32(_E), eid)

    acc = jnp.dot(feat.astype(jnp.bfloat16), W_all_ref[idx],
                  preferred_element_type=jnp.float32)
    out_ref[0] = jnp.maximum(acc, 0.0) + skip


@jax.jit
def kernel(x, W_patch, W_skip, W_desc, W_router, W_experts, W_fallback):
    W_ps_bf = jnp.concatenate([W_patch, W_skip],
                              axis=1).astype(jnp.bfloat16)
    W_all = jnp.concatenate(
        [W_experts, W_fallback[None]], 0).astype(jnp.bfloat16)

    grid_spec = dict(
        grid=(_B,),
        in_specs=[
            pl.BlockSpec((1, _C_IN, _H, _W), lambda b: (b, 0, 0, 0)),
            pl.BlockSpec((_CP, 2 * _C), lambda b: (0, 0)),
            pl.BlockSpec((_C, _C), lambda b: (0, 0)),
            pl.BlockSpec((_C, _E), lambda b: (0, 0)),
            pl.BlockSpec((_E + 1, _C, _C), lambda b: (0, 0, 0)),
        ],
        out_specs=pl.BlockSpec((1, _N_TOK, _C), lambda b: (b, 0, 0)),
    )
    return pl.pallas_call(
        _moe_step,
        out_shape=jax.ShapeDtypeStruct((_B, _N_TOK, _C), jnp.float32),
        **grid_spec,
    )(x, W_ps_bf, W_desc, W_router, W_all)


# speedup vs baseline: 1.6198x; 1.0851x over previous
import jax
import jax.numpy as jnp
from jax.experimental import pallas as pl
from jax.experimental.pallas import tpu as pltpu

_B = 8
_C_IN = 3
_H = 384
_W = 384
_P = 16
_N_TOK = (_H // _P) * (_W // _P)
_CP = _C_IN * _P * _P
_C = 384
_E = 4
_THR = 0.1


def _moe_step(x_ref, W_patch_ref, W_skip_ref, W_desc_ref, W_router_ref,
              W_experts_ref, W_fallback_ref, out_ref, wps_ref, wall_ref):
    i = pl.program_id(0)

    @pl.when(i == 0)
    def _prep():
        wps_ref[:, :_C] = W_patch_ref[...].astype(jnp.bfloat16)
        wps_ref[:, _C:] = W_skip_ref[...].astype(jnp.bfloat16)
        for e in range(_E):
            wall_ref[e] = W_experts_ref[e].astype(jnp.bfloat16)
        wall_ref[_E] = W_fallback_ref[...].astype(jnp.bfloat16)

    v = x_ref[0].astype(jnp.bfloat16).reshape(_C_IN, _H // _P, _P,
                                              _W // _P, _P)
    patches = v.transpose(1, 3, 0, 2, 4).reshape(_N_TOK, _CP)

    both = jnp.dot(patches, wps_ref[...],
                   preferred_element_type=jnp.float32)
    feat = jnp.tanh(both[:, :_C])
    skip = both[:, _C:]

    mean = jnp.mean(feat, axis=0, keepdims=True)
    desc = jnp.dot(mean, W_desc_ref[...],
                   preferred_element_type=jnp.float32)
    logits = jnp.dot(desc, W_router_ref[...],
                     preferred_element_type=jnp.float32)
    lmax = jnp.max(logits)
    conf = 1.0 / jnp.sum(jnp.exp(logits - lmax))
    eid = jnp.argmax(logits[0]).astype(jnp.int32)
    idx = jnp.where(conf < _THR, jnp.int32(_E), eid)

    acc = jnp.dot(feat.astype(jnp.bfloat16), wall_ref[idx],
                  preferred_element_type=jnp.float32)
    out_ref[0] = jnp.maximum(acc, 0.0) + skip


@jax.jit
def kernel(x, W_patch, W_skip, W_desc, W_router, W_experts, W_fallback):
    grid_spec = dict(
        grid=(_B,),
        in_specs=[
            pl.BlockSpec((1, _C_IN, _H, _W), lambda b: (b, 0, 0, 0)),
            pl.BlockSpec((_CP, _C), lambda b: (0, 0)),
            pl.BlockSpec((_CP, _C), lambda b: (0, 0)),
            pl.BlockSpec((_C, _C), lambda b: (0, 0)),
            pl.BlockSpec((_C, _E), lambda b: (0, 0)),
            pl.BlockSpec((_E, _C, _C), lambda b: (0, 0, 0)),
            pl.BlockSpec((_C, _C), lambda b: (0, 0)),
        ],
        out_specs=pl.BlockSpec((1, _N_TOK, _C), lambda b: (b, 0, 0)),
    )
    return pl.pallas_call(
        _moe_step,
        out_shape=jax.ShapeDtypeStruct((_B, _N_TOK, _C), jnp.float32),
        scratch_shapes=[
            pltpu.VMEM((_CP, 2 * _C), jnp.bfloat16),
            pltpu.VMEM((_E + 1, _C, _C), jnp.bfloat16),
        ],
        **grid_spec,
    )(x, W_patch, W_skip, W_desc, W_router, W_experts, W_fallback)
